# baseline (speedup 1.0000x reference)
import jax
import jax.numpy as jnp
from jax.experimental import pallas as pl


def _copy_body(in_ref, out_ref):
    out_ref[...] = in_ref[...]


def kernel(embeds):
    rows, cols = 500000, 128
    block_rows = 4000
    x = embeds.reshape(rows, cols)
    out = pl.pallas_call(
        _copy_body,
        grid=(rows // block_rows,),
        in_specs=[pl.BlockSpec((block_rows, cols), lambda i: (i, 0))],
        out_specs=pl.BlockSpec((block_rows, cols), lambda i: (i, 0)),
        out_shape=jax.ShapeDtypeStruct((rows, cols), jnp.float32),
    )(x)
    return out.reshape(embeds.shape)
